# Initial kernel scaffold; baseline (speedup 1.0000x reference)
import jax
import jax.numpy as jnp
from jax.experimental import pallas as pl


def kernel(opcode_probs, registers, k_write, q_read, gate, ln1_g, ln1_b, W1, b1, ln2_g, ln2_b, W2, b2, Wr, br, lnf_g, lnf_b):
    raise NotImplementedError("write your pallas kernel here")


# speedup vs baseline: 1.3887x; 1.3887x over previous
import functools

import jax
import jax.numpy as jnp
from jax.experimental import pallas as pl
from jax.experimental.pallas import tpu as pltpu

_B, _LINES, _CAT = 256, 32, 5
_NREG, _D = 64, 1024
_L, _K = 64, 16
_EPS = 1e-5
_BB = 32


def _roll(a, s):
    s = s % a.shape[-1]
    if s == 0:
        return a
    return jnp.concatenate([a[..., -s:], a[..., :-s]], axis=-1)


def _group_softmax_k(z):
    m = z
    for sh in (512, 256, 128, 64):
        m = jnp.maximum(m, _roll(m, sh))
    e = jnp.exp(z - m)
    s = e
    for sh in (512, 256, 128, 64):
        s = s + _roll(s, sh)
    return e / s


def _ln_last(x):
    m = jnp.mean(x, axis=-1, keepdims=True)
    xc = x - m
    v = jnp.mean(xc * xc, axis=-1, keepdims=True)
    return xc * jax.lax.rsqrt(v + _EPS)


def _tile16(chunk):
    return jnp.broadcast_to(chunk[:, None, :], (chunk.shape[0], 16, 64)).reshape(
        chunk.shape[0], 1024)


def _step_kernel(pg_ref, regs_in_ref, kw_ref, q_ref,
                 w1_ref, b1_ref, w2_ref, b2_ref, wr_ref, br_ref,
                 ln1g_ref, ln1b_ref, ln2g_ref, ln2b_ref,
                 lnfg_ref, lnfb_ref, out_ref):
    l = pl.program_id(1)

    @pl.when(l == 0)
    def _():
        out_ref[...] = regs_in_ref[...]

    regs = out_ref[...]
    regs2d = regs.reshape(_BB * _NREG, _D)

    q2 = q_ref[:, 0]
    kw = kw_ref[:, 0]
    qk = jnp.concatenate([q2, kw], axis=1).reshape(_BB * 3, _D)
    s_full = jax.lax.dot_general(
        regs2d, qk, ((( 1,), (1,)), ((), ())),
        preferred_element_type=jnp.float32)

    rows = jax.lax.broadcasted_iota(jnp.int32, (_BB * _NREG, _BB * 3), 0)
    cols = jax.lax.broadcasted_iota(jnp.int32, (_BB * _NREG, _BB * 3), 1)
    own = (cols // 3 == rows // _NREG)
    scale = 1.0 / (_D ** 0.5)
    s_own = jnp.where(own, s_full, 0.0) * scale
    j_id = cols % 3
    s0 = jnp.sum(jnp.where(j_id == 0, s_own, 0.0), axis=1, keepdims=True)
    s1 = jnp.sum(jnp.where(j_id == 1, s_own, 0.0), axis=1, keepdims=True)
    s2 = jnp.sum(jnp.where(j_id == 2, s_own, 0.0), axis=1, keepdims=True)
    s3 = jnp.concatenate([s0, s1, s2], axis=1)

    sg = s3.reshape(_BB, _NREG, 3)
    mg = jnp.max(sg, axis=1, keepdims=True)
    eg = jnp.exp(sg - mg)
    attn = eg / jnp.sum(eg, axis=1, keepdims=True)
    attn2d = attn.reshape(_BB * _NREG, 3)

    rows2 = jax.lax.broadcasted_iota(jnp.int32, (_BB * _NREG, _BB * 2), 0)
    cols2 = jax.lax.broadcasted_iota(jnp.int32, (_BB * _NREG, _BB * 2), 1)
    own2 = (cols2 // 2 == rows2 // _NREG)
    a0 = jnp.where(own2 & (cols2 % 2 == 0), attn2d[:, 0:1], 0.0)
    a1 = jnp.where(own2 & (cols2 % 2 == 1), attn2d[:, 1:2], 0.0)
    a_mat = a0 + a1
    op_s = jax.lax.dot_general(
        a_mat, regs2d, (((0,), (0,)), ((), ())),
        preferred_element_type=jnp.float32)
    op3 = op_s.reshape(_BB, 2, _D)

    h = _ln_last(op3)
    h1 = h[:, 0, :] * ln1g_ref[...] + ln1b_ref[...]
    h2 = h[:, 1, :] * ln2g_ref[...] + ln2b_ref[...]

    z1 = jnp.dot(h1, w1_ref[...], preferred_element_type=jnp.float32) + b1_ref[...]
    z2 = jnp.dot(h2, w2_ref[...], preferred_element_type=jnp.float32) + b2_ref[...]
    x = _group_softmax_k(z1)
    y = _group_softmax_k(z2)

    add = jnp.zeros_like(x)
    sub = jnp.zeros_like(x)
    for i in range(_K):
        xb = _tile16(x[:, i * 64:(i + 1) * 64])
        yb = _tile16(y[:, i * 64:(i + 1) * 64])
        add = add + xb * _roll(y, 64 * i)
        sub = sub + _roll(x, -64 * i) * yb

    pg = pg_ref[:, pl.ds(l, 1), :][:, 0, :]
    mix = (pg[:, 1:2] * x + pg[:, 2:3] * y +
           pg[:, 3:4] * add + pg[:, 4:5] * sub)

    v0 = jnp.dot(mix, wr_ref[...], preferred_element_type=jnp.float32) + br_ref[...]
    value = _ln_last(v0) * lnfg_ref[...] + lnfb_ref[...]

    geff = pg[:, 5:6] * (1.0 - pg[:, 0:1])
    w = attn[:, :, 2:3] * geff[:, None, :]
    out_ref[...] = regs + w * (value[:, None, :] - regs)


@jax.jit
def _run(pg, registers, k_write, q4, w1k, b1k, w2k, b2k, wrk, br2,
         ln1g, ln1b, ln2g, ln2b, lnfg, lnfb):
    nb = _B // _BB
    grid = (nb, _LINES)
    full = lambda *shape: shape
    return pl.pallas_call(
        _step_kernel,
        grid=grid,
        in_specs=[
            pl.BlockSpec((_BB, _LINES, 8), lambda i, l: (i, 0, 0)),
            pl.BlockSpec((_BB, _NREG, _D), lambda i, l: (i, 0, 0)),
            pl.BlockSpec((_BB, 1, 1, _D), lambda i, l: (i, l, 0, 0)),
            pl.BlockSpec((_BB, 1, 2, _D), lambda i, l: (i, l, 0, 0)),
            pl.BlockSpec((_D, _L * _K), lambda i, l: (0, 0)),
            pl.BlockSpec((1, _L * _K), lambda i, l: (0, 0)),
            pl.BlockSpec((_D, _L * _K), lambda i, l: (0, 0)),
            pl.BlockSpec((1, _L * _K), lambda i, l: (0, 0)),
            pl.BlockSpec((_L * _K, _D), lambda i, l: (0, 0)),
            pl.BlockSpec((1, _D), lambda i, l: (0, 0)),
            pl.BlockSpec((1, _D), lambda i, l: (0, 0)),
            pl.BlockSpec((1, _D), lambda i, l: (0, 0)),
            pl.BlockSpec((1, _D), lambda i, l: (0, 0)),
            pl.BlockSpec((1, _D), lambda i, l: (0, 0)),
            pl.BlockSpec((1, _D), lambda i, l: (0, 0)),
            pl.BlockSpec((1, _D), lambda i, l: (0, 0)),
        ],
        out_specs=pl.BlockSpec((_BB, _NREG, _D), lambda i, l: (i, 0, 0)),
        out_shape=jax.ShapeDtypeStruct((_B, _NREG, _D), jnp.float32),
        compiler_params=pltpu.CompilerParams(
            dimension_semantics=("parallel", "arbitrary"),
            vmem_limit_bytes=100 * 1024 * 1024,
        ),
    )(pg, registers, k_write, q4, w1k, b1k, w2k, b2k, wrk, br2,
      ln1g, ln1b, ln2g, ln2b, lnfg, lnfb)


def kernel(opcode_probs, registers, k_write, q_read, gate,
           ln1_g, ln1_b, W1, b1, ln2_g, ln2_b, W2, b2,
           Wr, br, lnf_g, lnf_b):
    f32 = jnp.float32
    pg = jnp.concatenate(
        [opcode_probs.astype(f32), gate.astype(f32),
         jnp.zeros((_B, _LINES, 2), f32)], axis=-1)
    q4 = q_read.astype(f32).reshape(_B, _LINES, 2, _D)
    w1k = W1.astype(f32).reshape(_D, _L, _K).transpose(0, 2, 1).reshape(_D, _L * _K)
    w2k = W2.astype(f32).reshape(_D, _L, _K).transpose(0, 2, 1).reshape(_D, _L * _K)
    b1k = b1.astype(f32).reshape(_L, _K).T.reshape(1, _L * _K)
    b2k = b2.astype(f32).reshape(_L, _K).T.reshape(1, _L * _K)
    wrk = Wr.astype(f32).reshape(_L, _K, _D).transpose(1, 0, 2).reshape(_L * _K, _D)
    br2 = br.astype(f32).reshape(1, _D)
    return _run(pg, registers.astype(f32), k_write.astype(f32), q4,
                w1k, b1k, w2k, b2k, wrk, br2,
                ln1_g.astype(f32).reshape(1, _D), ln1_b.astype(f32).reshape(1, _D),
                ln2_g.astype(f32).reshape(1, _D), ln2_b.astype(f32).reshape(1, _D),
                lnf_g.astype(f32).reshape(1, _D), lnf_b.astype(f32).reshape(1, _D))
